# Initial kernel scaffold; baseline (speedup 1.0000x reference)
import functools

import jax
import jax.numpy as jnp
from jax import lax
from jax.experimental import pallas as pl
from jax.experimental.pallas import tpu as pltpu
from jax.experimental.pallas import tpu_sc as plsc

NC = 2
NS = 16
LANES = 16
CH = 128
K = 80
ZR = 125


def _edge_agg(n_nodes: int, n_edges: int, interpret: bool = False):
    assert n_edges % K == 0 and n_nodes % (NS * ZR) == 0
    num_chunks = n_edges // K
    iters = (num_chunks + NS - 1) // NS
    rpt = n_nodes // NS

    def body(h_hbm, srco_hbm, dst_hbm, ea_hbm, out_hbm,
             src_v, dst_v, ea_v, rows_v, zbuf, acc, sem):
        c = lax.axis_index("c")
        s = lax.axis_index("s")

        zero16 = jnp.zeros((LANES,), jnp.float32)

        def zrow(r, carry):
            for cc in range(CH // LANES):
                zbuf[r, pl.ds(cc * LANES, LANES)] = zero16
            return carry

        lax.fori_loop(0, ZR, zrow, 0)
        for t in range(rpt // ZR):
            pltpu.sync_copy(zbuf, acc.at[pl.ds(s * rpt + t * ZR, ZR)])
        plsc.subcore_barrier()

        def chunk_body(t, carry):
            chunk = t * NS + s

            @pl.when(chunk < num_chunks)
            def _():
                ebase = pl.multiple_of(chunk * K, 8)
                sbase = pl.multiple_of(c * n_edges + chunk * K, 8)
                pltpu.sync_copy(srco_hbm.at[pl.ds(sbase, K)], src_v)
                pltpu.sync_copy(dst_hbm.at[pl.ds(ebase, K)], dst_v)
                pltpu.sync_copy(ea_hbm.at[pl.ds(ebase, K)], ea_v)
                pltpu.async_copy(h_hbm.at[src_v], rows_v, sem).wait()

                def edge(e, cy):
                    w16 = plsc.load_gather(
                        ea_v, [jnp.full((LANES,), e, jnp.int32)])
                    for cc in range(CH // LANES):
                        sl = pl.ds(cc * LANES, LANES)
                        rows_v[e, sl] = rows_v[e, sl] * w16
                    return cy

                lax.fori_loop(0, K, edge, 0)
                pltpu.sync_copy(rows_v, acc.at[dst_v], add=True)

            return carry

        lax.fori_loop(0, iters, chunk_body, 0)
        plsc.subcore_barrier()
        pltpu.sync_copy(acc.at[pl.ds(s * rpt, rpt)],
                        out_hbm.at[pl.ds(c * n_nodes + s * rpt, rpt)])

    mesh = plsc.VectorSubcoreMesh(
        core_axis_name="c", subcore_axis_name="s",
        num_cores=NC, num_subcores=NS)
    return pl.kernel(
        body,
        out_type=jax.ShapeDtypeStruct((NC * n_nodes, CH), jnp.float32),
        mesh=mesh,
        scratch_types=[
            pltpu.VMEM((K,), jnp.int32),
            pltpu.VMEM((K,), jnp.int32),
            pltpu.VMEM((K,), jnp.float32),
            pltpu.VMEM((K, CH), jnp.float32),
            pltpu.VMEM((ZR, CH), jnp.float32),
            pltpu.VMEM_SHARED((n_nodes, CH), jnp.float32),
            pltpu.SemaphoreType.DMA,
        ],
        interpret=interpret,
    )


def _mm_relu(h2, agg2, w):
    n = h2.shape[1]
    bm = 400
    assert n % bm == 0

    def body(h_ref, a_ref, w_ref, o_ref):
        hh = jnp.concatenate([h_ref[0], h_ref[1]], axis=1)
        aa = jnp.concatenate([a_ref[0], a_ref[1]], axis=1)
        r = jnp.dot(hh + aa, w_ref[...], preferred_element_type=jnp.float32)
        r = jnp.maximum(r, 0.0)
        o_ref[0] = r[:, :CH]
        o_ref[1] = r[:, CH:]

    return pl.pallas_call(
        body,
        grid=(n // bm,),
        in_specs=[
            pl.BlockSpec((2, bm, CH), lambda i: (0, i, 0)),
            pl.BlockSpec((2, bm, CH), lambda i: (0, i, 0)),
            pl.BlockSpec((2 * CH, 2 * CH), lambda i: (0, 0)),
        ],
        out_specs=pl.BlockSpec((2, bm, CH), lambda i: (0, i, 0)),
        out_shape=jax.ShapeDtypeStruct((2, n, CH), jnp.float32),
    )(h2, agg2, w)


def kernel(x, edge_index, edge_attr, pool_edge_index, pool_edge_attr,
           w1, w2, n_fine):
    n, c_full = x.shape
    e = edge_index.shape[1]
    ep = pool_edge_index.shape[1]

    dst = edge_index[0].astype(jnp.int32)
    src = edge_index[1].astype(jnp.int32)
    pdst = pool_edge_index[0].astype(jnp.int32)
    psrc = pool_edge_index[1].astype(jnp.int32)
    src_off = jnp.concatenate([src, src + n])
    psrc_off = jnp.concatenate([psrc, psrc + n])

    x2 = jnp.stack([x[:, :CH], x[:, CH:]])
    agg = _edge_agg(n, e)
    a1 = agg(x2.reshape(NC * n, CH), src_off, dst, edge_attr)
    h1 = _mm_relu(x2, a1.reshape(NC, n, CH), w1)
    a2 = agg(h1.reshape(NC * n, CH), src_off, dst, edge_attr)
    h2 = _mm_relu(h1, a2.reshape(NC, n, CH), w2)

    unpool = _edge_agg(n, ep)
    u = unpool(h2.reshape(NC * n, CH), psrc_off, pdst, pool_edge_attr)
    u2 = u.reshape(NC, n, CH)
    ufull = jnp.concatenate([u2[0], u2[1]], axis=1)
    out = jnp.zeros((n_fine, c_full), jnp.float32)
    return lax.dynamic_update_slice(out, ufull, (0, 0))


# speedup vs baseline: 2.6807x; 2.6807x over previous
import functools

import jax
import jax.numpy as jnp
from jax import lax
from jax.experimental import pallas as pl
from jax.experimental.pallas import tpu as pltpu
from jax.experimental.pallas import tpu_sc as plsc

NC = 2
NS = 16
LANES = 16
CH = 128
K = 80
ZR = 128
NF_OUT = 40000
ALIGN = NS * 8


def _edge_agg(n_nodes: int, n_edges: int, interpret: bool = False):
    assert n_edges % K == 0 and n_nodes % ALIGN == 0
    num_chunks = n_edges // K
    iters = (num_chunks + NS - 1) // NS
    rpt = n_nodes // NS

    def body(h_hbm, srco_hbm, dst_hbm, ea_hbm, out_hbm,
             src_v, dst_v, ea_v, rows_v, zbuf, acc, sem):
        c = lax.axis_index("c")
        s = lax.axis_index("s")

        zero16 = jnp.zeros((LANES,), jnp.float32)

        def zrow(r, carry):
            for cc in range(CH // LANES):
                zbuf[r, pl.ds(cc * LANES, LANES)] = zero16
            return carry

        lax.fori_loop(0, ZR, zrow, 0)
        for t in range(rpt // ZR):
            pltpu.sync_copy(zbuf, acc.at[pl.ds(s * rpt + t * ZR, ZR)])
        rem = rpt % ZR
        if rem:
            pltpu.sync_copy(
                zbuf.at[pl.ds(0, rem)],
                acc.at[pl.ds(s * rpt + (rpt // ZR) * ZR, rem)])
        plsc.subcore_barrier()

        def chunk_body(t, carry):
            chunk = t * NS + s

            @pl.when(chunk < num_chunks)
            def _():
                ebase = pl.multiple_of(chunk * K, 8)
                sbase = pl.multiple_of(c * n_edges + chunk * K, 8)
                pltpu.sync_copy(srco_hbm.at[pl.ds(sbase, K)], src_v)
                pltpu.sync_copy(dst_hbm.at[pl.ds(ebase, K)], dst_v)
                pltpu.sync_copy(ea_hbm.at[pl.ds(ebase, K)], ea_v)
                pltpu.async_copy(h_hbm.at[src_v], rows_v, sem).wait()

                def grp(g, cy):
                    ea16 = ea_v[pl.ds(pl.multiple_of(g * LANES, 8), LANES)]
                    for j in range(LANES):
                        w = ea16[j]
                        e = g * LANES + j
                        for cc in range(CH // LANES):
                            sl = pl.ds(cc * LANES, LANES)
                            rows_v[e, sl] = rows_v[e, sl] * w
                    return cy

                lax.fori_loop(0, K // LANES, grp, 0)
                pltpu.sync_copy(rows_v, acc.at[dst_v], add=True)

            return carry

        lax.fori_loop(0, iters, chunk_body, 0)
        plsc.subcore_barrier()
        pltpu.sync_copy(acc.at[pl.ds(s * rpt, rpt)],
                        out_hbm.at[pl.ds(c * n_nodes + s * rpt, rpt)])

    mesh = plsc.VectorSubcoreMesh(
        core_axis_name="c", subcore_axis_name="s",
        num_cores=NC, num_subcores=NS)
    return pl.kernel(
        body,
        out_type=jax.ShapeDtypeStruct((NC * n_nodes, CH), jnp.float32),
        mesh=mesh,
        scratch_types=[
            pltpu.VMEM((K,), jnp.int32),
            pltpu.VMEM((K,), jnp.int32),
            pltpu.VMEM((K,), jnp.float32),
            pltpu.VMEM((K, CH), jnp.float32),
            pltpu.VMEM((ZR, CH), jnp.float32),
            pltpu.VMEM_SHARED((n_nodes, CH), jnp.float32),
            pltpu.SemaphoreType.DMA,
        ],
        interpret=interpret,
    )


def _mm_relu(h2, agg2, w):
    n = h2.shape[1]
    bm = next(n // g for g in range(16, 256)
              if n % g == 0 and (n // g) % 8 == 0)

    def body(h_ref, a_ref, w_ref, o_ref):
        hh = jnp.concatenate([h_ref[0], h_ref[1]], axis=1)
        aa = jnp.concatenate([a_ref[0], a_ref[1]], axis=1)
        r = jnp.dot(hh + aa, w_ref[...], preferred_element_type=jnp.float32)
        r = jnp.maximum(r, 0.0)
        o_ref[0] = r[:, :CH]
        o_ref[1] = r[:, CH:]

    return pl.pallas_call(
        body,
        grid=(n // bm,),
        in_specs=[
            pl.BlockSpec((2, bm, CH), lambda i: (0, i, 0)),
            pl.BlockSpec((2, bm, CH), lambda i: (0, i, 0)),
            pl.BlockSpec((2 * CH, 2 * CH), lambda i: (0, 0)),
        ],
        out_specs=pl.BlockSpec((2, bm, CH), lambda i: (0, i, 0)),
        out_shape=jax.ShapeDtypeStruct((2, n, CH), jnp.float32),
    )(h2, agg2, w)


def kernel(x, edge_index, edge_attr, pool_edge_index, pool_edge_attr,
           w1, w2, n_fine):
    n, c_full = x.shape
    e = edge_index.shape[1]
    ep = pool_edge_index.shape[1]
    np_ = -(-n // ALIGN) * ALIGN

    dst = edge_index[0].astype(jnp.int32)
    src = edge_index[1].astype(jnp.int32)
    pdst = pool_edge_index[0].astype(jnp.int32)
    psrc = pool_edge_index[1].astype(jnp.int32)
    src_off = jnp.concatenate([src, src + np_])
    psrc_off = jnp.concatenate([psrc, psrc + np_])

    x2 = jnp.pad(jnp.stack([x[:, :CH], x[:, CH:]]),
                 ((0, 0), (0, np_ - n), (0, 0)))
    agg = _edge_agg(np_, e)
    a1 = agg(x2.reshape(NC * np_, CH), src_off, dst, edge_attr)
    h1 = _mm_relu(x2, a1.reshape(NC, np_, CH), w1)
    a2 = agg(h1.reshape(NC * np_, CH), src_off, dst, edge_attr)
    h2 = _mm_relu(h1, a2.reshape(NC, np_, CH), w2)

    unpool = _edge_agg(np_, ep)
    u = unpool(h2.reshape(NC * np_, CH), psrc_off, pdst, pool_edge_attr)
    u2 = u.reshape(NC, np_, CH)
    ufull = jnp.concatenate([u2[0, :n], u2[1, :n]], axis=1)
    out = jnp.zeros((NF_OUT, c_full), jnp.float32)
    return lax.dynamic_update_slice(out, ufull, (0, 0))
